# baseline (speedup 1.0000x reference)
import functools
import jax
import jax.numpy as jnp
from jax import lax
from jax.experimental import pallas as pl
from jax.experimental.pallas import tpu as pltpu
from jax.experimental.pallas import tpu_sc as plsc

N_NODES = 10000
N_EDGES = 320000
DX = 128
DY = 64
DT = 16

NC = 2
NS = 16
EPT = N_EDGES // NS
CHUNK = 40
NBUF = 8
NCHUNK = EPT // CHUNK
STEADY = NCHUNK - NCHUNK % NBUF
N_PAD = 10240
ROWS_PT = N_PAD // NS
ZROWS = 32


def _zero_vmem(ref, rows, width):
    def body(r, c):
        for j in range(width // 16):
            ref[r, pl.ds(j * 16, 16)] = jnp.zeros((16,), jnp.float32)
        return c
    lax.fori_loop(0, rows, body, 0)


def _aggregate_half(table_hbm, out_hbm, src_hbm, dst_hbm,
                    sidx, didx, buf, zbuf, acc, semi, semg, sid):
    my_rows = sid * ROWS_PT

    for k in range(ROWS_PT // ZROWS):
        pltpu.sync_copy(zbuf, acc.at[pl.ds(my_rows + k * ZROWS, ZROWS)])
    plsc.subcore_barrier()

    ebase = sid * EPT

    def idx_load(i, b):
        base = pl.multiple_of(ebase + i * CHUNK, 8)
        pltpu.async_copy(src_hbm.at[pl.ds(base, CHUNK)], sidx[b], semi[b])
        pltpu.async_copy(dst_hbm.at[pl.ds(base, CHUNK)], didx[b], semi[b])

    def idx_wait(i, b):
        base = pl.multiple_of(ebase + i * CHUNK, 8)
        pltpu.make_async_copy(src_hbm.at[pl.ds(base, CHUNK)], sidx[b],
                              semi[b]).wait()
        pltpu.make_async_copy(dst_hbm.at[pl.ds(base, CHUNK)], didx[b],
                              semi[b]).wait()

    def gather_start(b):
        pltpu.async_copy(table_hbm.at[sidx[b]], buf[b], semg[b])

    def gather_wait(b):
        pltpu.make_async_copy(table_hbm.at[sidx[b]], buf[b], semg[b]).wait()

    for j in range(NBUF):
        idx_load(j, j)
    for j in range(NBUF - 1):
        idx_wait(j, j)
        gather_start(j)

    @pl.loop(0, STEADY, step=NBUF)
    def _(g):
        for b in range(NBUF):
            i = g + b
            nb = (b + NBUF - 1) % NBUF
            gather_wait(b)
            pltpu.sync_copy(buf[b], acc.at[didx[b]], add=True)

            @pl.when(i + NBUF - 1 < NCHUNK)
            def _():
                idx_wait(i + NBUF - 1, nb)
                gather_start(nb)

            @pl.when(i + NBUF < NCHUNK)
            def _():
                idx_load(i + NBUF, b)

    for r in range(STEADY, NCHUNK):
        b = r % NBUF
        gather_wait(b)
        pltpu.sync_copy(buf[b], acc.at[didx[b]], add=True)

    plsc.subcore_barrier()

    pltpu.sync_copy(acc.at[pl.ds(my_rows, ROWS_PT)],
                    out_hbm.at[pl.ds(my_rows, ROWS_PT)])


def _sc_aggregate_body(src_hbm, dst_hbm, hX_hbm, hYp_hbm, pX_hbm, pY_hbm,
                       *scratch):
    cid = lax.axis_index("c")
    sid = lax.axis_index("s")
    sidx = list(scratch[0:NBUF])
    didx = list(scratch[NBUF:2 * NBUF])
    buf = list(scratch[2 * NBUF:3 * NBUF])
    zbuf = scratch[3 * NBUF]
    acc = scratch[3 * NBUF + 1]
    semi = list(scratch[3 * NBUF + 2:3 * NBUF + 2 + NBUF])
    semg = list(scratch[3 * NBUF + 2 + NBUF:3 * NBUF + 2 + 2 * NBUF])

    _zero_vmem(zbuf, ZROWS, DX)

    @pl.when(cid == 0)
    def _():
        _aggregate_half(hX_hbm, pX_hbm, src_hbm, dst_hbm,
                        sidx, didx, buf, zbuf, acc, semi, semg, sid)

    @pl.when(cid == 1)
    def _():
        _aggregate_half(hYp_hbm, pY_hbm, src_hbm, dst_hbm,
                        sidx, didx, buf, zbuf, acc, semi, semg, sid)


def _sc_aggregate(src, dst, h_X, h_Yp):
    mesh = plsc.VectorSubcoreMesh(core_axis_name="c", subcore_axis_name="s")
    fn = pl.kernel(
        _sc_aggregate_body,
        out_type=(
            jax.ShapeDtypeStruct((N_PAD, DX), jnp.float32),
            jax.ShapeDtypeStruct((N_PAD, DX), jnp.float32),
        ),
        mesh=mesh,
        scratch_types=(
            [pltpu.VMEM((CHUNK,), jnp.int32)] * (2 * NBUF)
            + [pltpu.VMEM((CHUNK, DX), jnp.float32)] * NBUF
            + [
                pltpu.VMEM((ZROWS, DX), jnp.float32),
                pltpu.VMEM_SHARED((N_PAD, DX), jnp.float32),
            ]
            + [pltpu.SemaphoreType.DMA] * (2 * NBUF)
        ),
    )
    return fn(src, dst, h_X, h_Yp)


def _ln(v, g, b):
    mu = jnp.mean(v, axis=-1, keepdims=True)
    var = jnp.mean((v - mu) * (v - mu), axis=-1, keepdims=True)
    return (v - mu) * lax.rsqrt(var + 1e-5) * g + b


def _tc_update_body(pX, pY, ht, wxt, bx, gx, bex, wyt, by, gy, bey, ox, oy):
    aX = pX[...]
    aY = pY[:, :DY]
    dot = functools.partial(jnp.dot, preferred_element_type=jnp.float32,
                            precision=lax.Precision.HIGHEST)
    t = dot(ht[...], wxt[DX + DY:])
    xb = dot(aX, wxt[:DX]) + dot(aY, wxt[DX:DX + DY]) + t + bx[...]
    xb = jnp.maximum(xb, 0.0)
    ox[...] = _ln(xb, gx[...], bex[...])
    yb = dot(aY, wyt[...]) + by[...]
    yb = jnp.maximum(yb, 0.0)
    oy[...] = _ln(yb, gy[...], bey[...])


def _tc_update(pX, pY, h_t, WXT, bX, gX, betX, WYT, bY, gY, betY):
    R = 2000
    grid = (N_NODES // R,)
    full = lambda shape: pl.BlockSpec(shape, lambda i: (0,) * len(shape))
    return pl.pallas_call(
        _tc_update_body,
        grid=grid,
        in_specs=[
            pl.BlockSpec((R, DX), lambda i: (i, 0)),
            pl.BlockSpec((R, DX), lambda i: (i, 0)),
            full((1, DT)),
            full((DX + DY + DT, DX)),
            full((1, DX)),
            full((1, DX)),
            full((1, DX)),
            full((DY, DY)),
            full((1, DY)),
            full((1, DY)),
            full((1, DY)),
        ],
        out_specs=[
            pl.BlockSpec((R, DX), lambda i: (i, 0)),
            pl.BlockSpec((R, DY), lambda i: (i, 0)),
        ],
        out_shape=[
            jax.ShapeDtypeStruct((N_NODES, DX), jnp.float32),
            jax.ShapeDtypeStruct((N_NODES, DY), jnp.float32),
        ],
    )(pX, pY, h_t, WXT, bX, gX, betX, WYT, bY, gY, betY)


def kernel(edge_index, h_X, h_Y, h_t, WX, bX, gX, betX, WY, bY, gY, betY):
    src = edge_index[0].astype(jnp.int32)
    dst = edge_index[1].astype(jnp.int32)
    h_Yp = jnp.pad(h_Y, ((0, 0), (0, DX - DY)))
    pX, pY = _sc_aggregate(src, dst, h_X, h_Yp)
    x, y = _tc_update(
        pX, pY, h_t, WX.T, bX.reshape(1, DX), gX.reshape(1, DX),
        betX.reshape(1, DX), WY.T, bY.reshape(1, DY), gY.reshape(1, DY),
        betY.reshape(1, DY))
    return (x, y)


# speedup vs baseline: 1.3459x; 1.3459x over previous
import functools
import jax
import jax.numpy as jnp
from jax import lax
from jax.experimental import pallas as pl
from jax.experimental.pallas import tpu as pltpu
from jax.experimental.pallas import tpu_sc as plsc

N_NODES = 10000
N_EDGES = 320000
DX = 128
DY = 64
DT = 16

NC = 2
NS = 16
EPT = N_EDGES // NS
CHUNK = 80
NBUF = 4
NCHUNK = EPT // CHUNK
STEADY = NCHUNK - NCHUNK % NBUF
N_PAD = 10240
ROWS_PT = N_PAD // NS
ZROWS = 32
assert EPT == NCHUNK * CHUNK


def _zero_vmem(ref, rows, width):
    def body(r, c):
        for j in range(width // 16):
            ref[r, pl.ds(j * 16, 16)] = jnp.zeros((16,), jnp.float32)
        return c
    lax.fori_loop(0, rows, body, 0)


def _aggregate_half(table_hbm, out_hbm, src_hbm, dst_hbm,
                    sidx, didx, buf, zbuf, acc, semi, semg, sid):
    my_rows = sid * ROWS_PT
    ebase = sid * EPT

    def idx_load(i, b):
        base = pl.multiple_of(ebase + i * CHUNK, 8)
        pltpu.async_copy(src_hbm.at[pl.ds(base, CHUNK)], sidx[b], semi[b])
        pltpu.async_copy(dst_hbm.at[pl.ds(base, CHUNK)], didx[b], semi[b])

    def idx_wait(i, b):
        base = pl.multiple_of(ebase + i * CHUNK, 8)
        pltpu.make_async_copy(src_hbm.at[pl.ds(base, CHUNK)], sidx[b],
                              semi[b]).wait()
        pltpu.make_async_copy(dst_hbm.at[pl.ds(base, CHUNK)], didx[b],
                              semi[b]).wait()

    def gather_start(b):
        pltpu.async_copy(table_hbm.at[sidx[b]], buf[b], semg[b])

    def gather_wait(b):
        pltpu.make_async_copy(table_hbm.at[sidx[b]], buf[b], semg[b]).wait()

    for j in range(NBUF):
        idx_load(j, j)
    for j in range(NBUF - 1):
        idx_wait(j, j)
        gather_start(j)

    for k in range(ROWS_PT // ZROWS):
        pltpu.sync_copy(zbuf, acc.at[pl.ds(my_rows + k * ZROWS, ZROWS)])
    plsc.subcore_barrier()

    @pl.loop(0, STEADY, step=NBUF)
    def _(g):
        for b in range(NBUF):
            i = g + b
            nb = (b + NBUF - 1) % NBUF
            gather_wait(b)
            pltpu.sync_copy(buf[b], acc.at[didx[b]], add=True)

            @pl.when(i + NBUF - 1 < NCHUNK)
            def _():
                idx_wait(i + NBUF - 1, nb)
                gather_start(nb)

            @pl.when(i + NBUF < NCHUNK)
            def _():
                idx_load(i + NBUF, b)

    for r in range(STEADY, NCHUNK):
        b = r % NBUF
        gather_wait(b)
        pltpu.sync_copy(buf[b], acc.at[didx[b]], add=True)

    plsc.subcore_barrier()

    pltpu.sync_copy(acc.at[pl.ds(my_rows, ROWS_PT)],
                    out_hbm.at[pl.ds(my_rows, ROWS_PT)])


def _sc_aggregate_body(src_hbm, dst_hbm, hX_hbm, hYp_hbm, pX_hbm, pY_hbm,
                       *scratch):
    cid = lax.axis_index("c")
    sid = lax.axis_index("s")
    sidx = list(scratch[0:NBUF])
    didx = list(scratch[NBUF:2 * NBUF])
    buf = list(scratch[2 * NBUF:3 * NBUF])
    zbuf = scratch[3 * NBUF]
    acc = scratch[3 * NBUF + 1]
    semi = list(scratch[3 * NBUF + 2:3 * NBUF + 2 + NBUF])
    semg = list(scratch[3 * NBUF + 2 + NBUF:3 * NBUF + 2 + 2 * NBUF])

    _zero_vmem(zbuf, ZROWS, DX)

    @pl.when(cid == 0)
    def _():
        _aggregate_half(hX_hbm, pX_hbm, src_hbm, dst_hbm,
                        sidx, didx, buf, zbuf, acc, semi, semg, sid)

    @pl.when(cid == 1)
    def _():
        _aggregate_half(hYp_hbm, pY_hbm, src_hbm, dst_hbm,
                        sidx, didx, buf, zbuf, acc, semi, semg, sid)


def _sc_aggregate(src, dst, h_X, h_Yp):
    mesh = plsc.VectorSubcoreMesh(core_axis_name="c", subcore_axis_name="s")
    fn = pl.kernel(
        _sc_aggregate_body,
        out_type=(
            jax.ShapeDtypeStruct((N_PAD, DX), jnp.float32),
            jax.ShapeDtypeStruct((N_PAD, DX), jnp.float32),
        ),
        mesh=mesh,
        scratch_types=(
            [pltpu.VMEM((CHUNK,), jnp.int32)] * (2 * NBUF)
            + [pltpu.VMEM((CHUNK, DX), jnp.float32)] * NBUF
            + [
                pltpu.VMEM((ZROWS, DX), jnp.float32),
                pltpu.VMEM_SHARED((N_PAD, DX), jnp.float32),
            ]
            + [pltpu.SemaphoreType.DMA] * (2 * NBUF)
        ),
    )
    return fn(src, dst, h_X, h_Yp)


def _ln(v, g, b):
    mu = jnp.mean(v, axis=-1, keepdims=True)
    var = jnp.mean((v - mu) * (v - mu), axis=-1, keepdims=True)
    return (v - mu) * lax.rsqrt(var + 1e-5) * g + b


def _tc_update_body(pX, pY, ht, wxt, bx, gx, bex, wyt, by, gy, bey, ox, oy):
    aX = pX[...]
    aY = pY[:, :DY]
    dot = functools.partial(jnp.dot, preferred_element_type=jnp.float32,
                            precision=lax.Precision.HIGHEST)
    t = dot(ht[...], wxt[DX + DY:])
    xb = dot(aX, wxt[:DX]) + dot(aY, wxt[DX:DX + DY]) + t + bx[...]
    xb = jnp.maximum(xb, 0.0)
    ox[...] = _ln(xb, gx[...], bex[...])
    yb = dot(aY, wyt[...]) + by[...]
    yb = jnp.maximum(yb, 0.0)
    oy[...] = _ln(yb, gy[...], bey[...])


def _tc_update(pX, pY, h_t, WXT, bX, gX, betX, WYT, bY, gY, betY):
    R = 2000
    grid = (N_NODES // R,)
    full = lambda shape: pl.BlockSpec(shape, lambda i: (0,) * len(shape))
    return pl.pallas_call(
        _tc_update_body,
        grid=grid,
        in_specs=[
            pl.BlockSpec((R, DX), lambda i: (i, 0)),
            pl.BlockSpec((R, DX), lambda i: (i, 0)),
            full((1, DT)),
            full((DX + DY + DT, DX)),
            full((1, DX)),
            full((1, DX)),
            full((1, DX)),
            full((DY, DY)),
            full((1, DY)),
            full((1, DY)),
            full((1, DY)),
        ],
        out_specs=[
            pl.BlockSpec((R, DX), lambda i: (i, 0)),
            pl.BlockSpec((R, DY), lambda i: (i, 0)),
        ],
        out_shape=[
            jax.ShapeDtypeStruct((N_NODES, DX), jnp.float32),
            jax.ShapeDtypeStruct((N_NODES, DY), jnp.float32),
        ],
    )(pX, pY, h_t, WXT, bX, gX, betX, WYT, bY, gY, betY)


def kernel(edge_index, h_X, h_Y, h_t, WX, bX, gX, betX, WY, bY, gY, betY):
    src = edge_index[0].astype(jnp.int32)
    dst = edge_index[1].astype(jnp.int32)
    h_Yp = jnp.pad(h_Y, ((0, 0), (0, DX - DY)))
    pX, pY = _sc_aggregate(src, dst, h_X, h_Yp)
    x, y = _tc_update(
        pX, pY, h_t, WX.T, bX.reshape(1, DX), gX.reshape(1, DX),
        betX.reshape(1, DX), WY.T, bY.reshape(1, DY), gY.reshape(1, DY),
        betY.reshape(1, DY))
    return (x, y)
